# baseline (speedup 1.0000x reference)
import functools

import jax
import jax.numpy as jnp
from jax import lax
from jax.experimental import pallas as pl
from jax.experimental.pallas import tpu as pltpu
from jax.experimental.pallas import tpu_sc as plsc

_VOCAB = 100000
_D = 1024
_B = 4
_S = 2048
_N = _B * _S
_EPS = 1e-07

_NC = 2
_NS = 16
_NW = _NC * _NS
_K = 32
_NBUF = 3

_NSPLIT = 2
_BH = _B // _NSPLIT
_NH = _BH * _S
_TPW = _NH // _NW
_NCHUNK = _TPW // _K

_R = 512


def _sc_gather_body(ids_ref, wemb_ref, out_ref, idx_buf,
                    b0, b1, b2, gs0, gs1, gs2, ss0, ss1, ss2):
    bufs = (b0, b1, b2)
    gsems = (gs0, gs1, gs2)
    ssems = (ss0, ss1, ss2)
    wid = lax.axis_index("s") * _NC + lax.axis_index("c")
    t0 = wid * _TPW

    idx_handles = [
        pltpu.async_copy(ids_ref.at[pl.ds(t0 + c * _K, _K)], idx_buf.at[c], gs0)
        for c in range(_NCHUNK)
    ]
    for hd in idx_handles:
        hd.wait()

    def fire_gather(c):
        i = c % _NBUF
        return pltpu.async_copy(wemb_ref.at[idx_buf.at[c]], bufs[i], gsems[i])

    def fire_store(c):
        i = c % _NBUF
        return pltpu.async_copy(bufs[i], out_ref.at[pl.ds(t0 + c * _K, _K)], ssems[i])

    gh = {0: fire_gather(0), 1: fire_gather(1)}
    sh = {}
    for c in range(_NCHUNK):
        gh[c].wait()
        sh[c] = fire_store(c)
        nxt = c + 2
        if nxt < _NCHUNK:
            prev = nxt - _NBUF
            if prev >= 0:
                sh[prev].wait()
            gh[nxt] = fire_gather(nxt)
    sh[_NCHUNK - 2].wait()
    sh[_NCHUNK - 1].wait()


def _sc_gather(ids_flat, wemb):
    mesh = plsc.VectorSubcoreMesh(core_axis_name="c", subcore_axis_name="s")
    run = functools.partial(
        pl.kernel,
        mesh=mesh,
        compiler_params=pltpu.CompilerParams(needs_layout_passes=False),
        out_type=jax.ShapeDtypeStruct((_NH, _D), jnp.float32),
        scratch_types=[
            pltpu.VMEM((_NCHUNK, _K), jnp.int32),
            pltpu.VMEM((_K, _D), jnp.float32),
            pltpu.VMEM((_K, _D), jnp.float32),
            pltpu.VMEM((_K, _D), jnp.float32),
            pltpu.SemaphoreType.DMA,
            pltpu.SemaphoreType.DMA,
            pltpu.SemaphoreType.DMA,
            pltpu.SemaphoreType.DMA,
            pltpu.SemaphoreType.DMA,
            pltpu.SemaphoreType.DMA,
        ],
    )(_sc_gather_body)
    return run(ids_flat, wemb)


def _tc_ln_body(x_ref, p_ref, m_ref, g_ref, b_ref, o_ref):
    x = x_ref[...] + p_ref[...][None, :, :]
    s1 = jnp.sum(x, axis=2, keepdims=True)
    s2 = jnp.sum(x * x, axis=2, keepdims=True)
    mean = s1 * (1.0 / _D)
    var = s2 * (1.0 / _D) - mean * mean
    y = (x - mean) * lax.rsqrt(var + _EPS)
    o_ref[...] = (g_ref[...][None] * y + b_ref[...][None]) * m_ref[...]


def _tc_ln(gathered3d, pos, mask3d, gamma2d, beta2d):
    grid = (_S // _R,)
    return pl.pallas_call(
        _tc_ln_body,
        grid=grid,
        in_specs=[
            pl.BlockSpec((_BH, _R, _D), lambda i: (0, i, 0)),
            pl.BlockSpec((_R, _D), lambda i: (i, 0)),
            pl.BlockSpec((_BH, _R, 1), lambda i: (0, i, 0)),
            pl.BlockSpec((1, _D), lambda i: (0, 0)),
            pl.BlockSpec((1, _D), lambda i: (0, 0)),
        ],
        out_specs=pl.BlockSpec((_BH, _R, _D), lambda i: (0, i, 0)),
        out_shape=jax.ShapeDtypeStruct((_BH, _S, _D), jnp.float32),
    )(gathered3d, pos, mask3d, gamma2d, beta2d)


@jax.jit
def _run(ids_flat, mask3d, wemb, pos, gamma2d, beta2d):
    halves = []
    for h in range(_NSPLIT):
        ids_h = lax.dynamic_slice_in_dim(ids_flat, h * _NH, _NH)
        gathered = _sc_gather(ids_h, wemb)
        halves.append(gathered.reshape(_BH, _S, _D))
    outs = []
    for h in range(_NSPLIT):
        mask_h = lax.dynamic_slice_in_dim(mask3d, h * _BH, _BH)
        outs.append(_tc_ln(halves[h], pos, mask_h, gamma2d, beta2d))
    return jnp.concatenate(outs, axis=0)


def kernel(input_ids, mask, word_embeddings, position_embeddings, ln_gamma, ln_beta):
    return _run(
        input_ids.reshape(-1).astype(jnp.int32),
        mask.reshape(_B, _S, 1).astype(jnp.float32),
        word_embeddings,
        position_embeddings,
        ln_gamma.reshape(1, _D),
        ln_beta.reshape(1, _D),
    )


# speedup vs baseline: 1.3399x; 1.3399x over previous
import functools

import jax
import jax.numpy as jnp
from jax import lax
from jax.experimental import pallas as pl
from jax.experimental.pallas import tpu as pltpu
from jax.experimental.pallas import tpu_sc as plsc

_VOCAB = 100000
_D = 1024
_B = 4
_S = 2048
_N = _B * _S
_EPS = 1e-07

_NC = 2
_NS = 16
_NW = _NC * _NS
_TPW = _N // _NW
_K = 32
_NCHUNK = _TPW // _K
_NBUF = 3

_R = 256


def _sc_gather_body(ids_ref, wemb_ref, out_ref, idx_buf,
                    b0, b1, b2, gs0, gs1, gs2, ss0, ss1, ss2):
    bufs = (b0, b1, b2)
    gsems = (gs0, gs1, gs2)
    ssems = (ss0, ss1, ss2)
    wid = lax.axis_index("s") * _NC + lax.axis_index("c")
    t0 = wid * _TPW

    idx_handles = [
        pltpu.async_copy(ids_ref.at[pl.ds(t0 + c * _K, _K)], idx_buf.at[c], gs0)
        for c in range(_NCHUNK)
    ]
    for hd in idx_handles:
        hd.wait()

    def fire_gather(c):
        i = c % _NBUF
        return pltpu.async_copy(wemb_ref.at[idx_buf.at[c]], bufs[i], gsems[i])

    def fire_store(c):
        i = c % _NBUF
        return pltpu.async_copy(bufs[i], out_ref.at[pl.ds(t0 + c * _K, _K)], ssems[i])

    gh = {0: fire_gather(0), 1: fire_gather(1)}
    sh = {}
    for c in range(_NCHUNK):
        gh[c].wait()
        sh[c] = fire_store(c)
        nxt = c + 2
        if nxt < _NCHUNK:
            prev = nxt - _NBUF
            if prev >= 0:
                sh[prev].wait()
            gh[nxt] = fire_gather(nxt)
    sh[_NCHUNK - 2].wait()
    sh[_NCHUNK - 1].wait()


def _sc_gather(ids_flat, wemb):
    mesh = plsc.VectorSubcoreMesh(core_axis_name="c", subcore_axis_name="s")
    run = functools.partial(
        pl.kernel,
        mesh=mesh,
        compiler_params=pltpu.CompilerParams(needs_layout_passes=False),
        out_type=jax.ShapeDtypeStruct((_N, _D), jnp.float32),
        scratch_types=[
            pltpu.VMEM((_NCHUNK, _K), jnp.int32),
            pltpu.VMEM((_K, _D), jnp.float32),
            pltpu.VMEM((_K, _D), jnp.float32),
            pltpu.VMEM((_K, _D), jnp.float32),
            pltpu.SemaphoreType.DMA,
            pltpu.SemaphoreType.DMA,
            pltpu.SemaphoreType.DMA,
            pltpu.SemaphoreType.DMA,
            pltpu.SemaphoreType.DMA,
            pltpu.SemaphoreType.DMA,
        ],
    )(_sc_gather_body)
    return run(ids_flat, wemb)


def _tc_ln_body(x_ref, p_ref, m_ref, g_ref, b_ref, o_ref):
    x = x_ref[...] + p_ref[...][None, :, :]
    s1 = jnp.sum(x, axis=2, keepdims=True)
    s2 = jnp.sum(x * x, axis=2, keepdims=True)
    mean = s1 * (1.0 / _D)
    var = s2 * (1.0 / _D) - mean * mean
    y = (x - mean) * lax.rsqrt(var + _EPS)
    o_ref[...] = (g_ref[...][None] * y + b_ref[...][None]) * m_ref[...]


def _tc_ln(gathered3d, pos, mask3d, gamma2d, beta2d):
    grid = (_S // _R,)
    return pl.pallas_call(
        _tc_ln_body,
        grid=grid,
        in_specs=[
            pl.BlockSpec((_B, _R, _D), lambda i: (0, i, 0)),
            pl.BlockSpec((_R, _D), lambda i: (i, 0)),
            pl.BlockSpec((_B, _R, 1), lambda i: (0, i, 0)),
            pl.BlockSpec((1, _D), lambda i: (0, 0)),
            pl.BlockSpec((1, _D), lambda i: (0, 0)),
        ],
        out_specs=pl.BlockSpec((_B, _R, _D), lambda i: (0, i, 0)),
        out_shape=jax.ShapeDtypeStruct((_B, _S, _D), jnp.float32),
    )(gathered3d, pos, mask3d, gamma2d, beta2d)


@jax.jit
def _run(ids_flat, mask3d, wemb, pos, gamma2d, beta2d):
    gathered = _sc_gather(ids_flat, wemb)
    return _tc_ln(gathered.reshape(_B, _S, _D), pos, mask3d, gamma2d, beta2d)


def kernel(input_ids, mask, word_embeddings, position_embeddings, ln_gamma, ln_beta):
    return _run(
        input_ids.reshape(-1).astype(jnp.int32),
        mask.reshape(_B, _S, 1).astype(jnp.float32),
        word_embeddings,
        position_embeddings,
        ln_gamma.reshape(1, _D),
        ln_beta.reshape(1, _D),
    )
